# Initial kernel scaffold; baseline (speedup 1.0000x reference)
import jax
import jax.numpy as jnp
from jax.experimental import pallas as pl


def kernel(query, keys, values, query_idx, WQ, bQ, WK, bK, WV, bV, WP, bP, a, prelu_w, gamma, beta):
    raise NotImplementedError("write your pallas kernel here")


# speedup vs baseline: 28.1671x; 28.1671x over previous
import jax
import jax.numpy as jnp
from jax import lax
from jax.experimental import pallas as pl
from jax.experimental.pallas import tpu as pltpu
from jax.experimental.pallas import tpu_sc as plsc

N = 10000
E = 320000
DIM = 128
H = 8
DH = DIM // H

SC_CORES = 2
SC_SUBCORES = 16
SC_TILES = SC_CORES * SC_SUBCORES
EPT = E // SC_TILES
CHUNK = 80
NCHUNK = EPT // CHUNK
RPT = 624
NREM = N - SC_SUBCORES * RPT
ND = 640
DPT = ND // SC_SUBCORES

BN = 1000
BE = 640


def _mesh():
    return plsc.VectorSubcoreMesh(
        core_axis_name="c", subcore_axis_name="s",
        num_cores=SC_CORES, num_subcores=SC_SUBCORES)


def _params():
    return pltpu.CompilerParams(needs_layout_passes=False)


def _sel():
    r = lax.broadcasted_iota(jnp.int32, (DIM, H), 0)
    c = lax.broadcasted_iota(jnp.int32, (DIM, H), 1)
    return (r // DH == c).astype(jnp.float32)


def _selT():
    r = lax.broadcasted_iota(jnp.int32, (H, DIM), 0)
    c = lax.broadcasted_iota(jnp.int32, (H, DIM), 1)
    return (c // DH == r).astype(jnp.float32)


def _tile8():
    r = lax.broadcasted_iota(jnp.int32, (H, DIM), 0)
    c = lax.broadcasted_iota(jnp.int32, (H, DIM), 1)
    return (c % H == r).astype(jnp.float32)


def _mm(x, w):
    return lax.dot_general(x, w, (((1,), (1,)), ((), ())),
                           preferred_element_type=jnp.float32)


def _mmn(x, w):
    return lax.dot_general(x, w, (((1,), (0,)), ((), ())),
                           preferred_element_type=jnp.float32)


def _node_body(q_ref, wq_ref, wk_ref, wv_ref, bq_ref, bk_ref, bv_ref,
               af_ref, pw_ref, qq_ref, qvw_ref, dq_ref):
    q = q_ref[...]
    qQ = _mm(q, wq_ref[...]) + bq_ref[...]
    qK = _mm(q, wk_ref[...]) + bk_ref[...]
    u = qQ + qK
    pw = pw_ref[0, 0]
    pr = jnp.where(u >= 0, u, pw * u)
    eq = jnp.exp(_mmn(pr * af_ref[...], _sel()))
    qq_ref[...] = qQ
    qV = _mm(q, wv_ref[...]) + bv_ref[...]
    qvw_ref[...] = qV * _mmn(eq, _selT())
    dq_ref[...] = eq


def _tc_node(query, WQ, WK, WV, bQ, bK, bV, aflat, pw):
    full = lambda s: pl.BlockSpec(s, lambda i: (0, 0))
    return pl.pallas_call(
        _node_body,
        grid=(N // BN,),
        in_specs=[
            pl.BlockSpec((BN, DIM), lambda i: (i, 0)),
            full((DIM, DIM)), full((DIM, DIM)), full((DIM, DIM)),
            full((1, DIM)), full((1, DIM)), full((1, DIM)),
            full((1, DIM)), full((1, 1)),
        ],
        out_specs=[
            pl.BlockSpec((BN, DIM), lambda i: (i, 0)),
            pl.BlockSpec((BN, DIM), lambda i: (i, 0)),
            pl.BlockSpec((BN, H), lambda i: (i, 0)),
        ],
        out_shape=[
            jax.ShapeDtypeStruct((N, DIM), jnp.float32),
            jax.ShapeDtypeStruct((N, DIM), jnp.float32),
            jax.ShapeDtypeStruct((N, H), jnp.float32),
        ],
    )(query, WQ, WK, WV, bQ, bK, bV, aflat, pw)


def _sc_gather_body(tab_hbm, idx_hbm, out_hbm, idx_v, buf):
    wid = lax.axis_index("s") * SC_CORES + lax.axis_index("c")
    base = pl.multiple_of(wid * EPT, 8)

    @pl.loop(0, NCHUNK)
    def _(i):
        off = pl.multiple_of(base + i * CHUNK, 8)
        pltpu.sync_copy(idx_hbm.at[pl.ds(off, CHUNK)], idx_v)
        pltpu.sync_copy(tab_hbm.at[idx_v], buf)
        pltpu.sync_copy(buf, out_hbm.at[pl.ds(off, CHUNK)])


def _sc_gather(qQ, query_idx):
    k = pl.kernel(
        _sc_gather_body,
        out_type=jax.ShapeDtypeStruct((E, DIM), jnp.float32),
        mesh=_mesh(),
        compiler_params=_params(),
        scratch_types=[
            pltpu.VMEM((CHUNK,), jnp.int32),
            pltpu.VMEM((CHUNK, DIM), jnp.float32),
        ],
    )
    return k(qQ, query_idx)


def _edge_body(k_ref, v_ref, g_ref, di_ref, wk_ref, wv_ref, bk_ref, bv_ref,
               af_ref, pw_ref, w_ref, d_ref):
    kK = _mm(k_ref[...], wk_ref[...]) + bk_ref[...]
    u = g_ref[...] + kK
    pw = pw_ref[0, 0]
    pr = jnp.where(u >= 0, u, pw * u)
    e = jnp.exp(_mmn(pr * af_ref[...], _sel()))
    vV = _mm(v_ref[...], wv_ref[...]) + bv_ref[...]
    w_ref[...] = vV * _mmn(e, _selT())
    col = lax.broadcasted_iota(jnp.int32, (BE, DIM), 1)
    slot = jnp.bitwise_and(di_ref[...], 15)
    mask = (lax.shift_right_logical(col, 3) == slot)
    d_ref[...] = _mmn(e, _tile8()) * mask.astype(jnp.float32)


def _tc_edge(keys, values, qa, idx2d, WK, WV, bK, bV, aflat, pw):
    full = lambda s: pl.BlockSpec(s, lambda i: (0, 0))
    return pl.pallas_call(
        _edge_body,
        grid=(E // BE,),
        in_specs=[
            pl.BlockSpec((BE, DIM), lambda i: (i, 0)),
            pl.BlockSpec((BE, DIM), lambda i: (i, 0)),
            pl.BlockSpec((BE, DIM), lambda i: (i, 0)),
            pl.BlockSpec((BE, 1), lambda i: (i, 0)),
            full((DIM, DIM)), full((DIM, DIM)),
            full((1, DIM)), full((1, DIM)),
            full((1, DIM)), full((1, 1)),
        ],
        out_specs=[
            pl.BlockSpec((BE, DIM), lambda i: (i, 0)),
            pl.BlockSpec((BE, DIM), lambda i: (i, 0)),
        ],
        out_shape=[
            jax.ShapeDtypeStruct((E, DIM), jnp.float32),
            jax.ShapeDtypeStruct((E, DIM), jnp.float32),
        ],
    )(keys, values, qa, idx2d, WK, WV, bK, bV, aflat, pw)


def _sc_scatter_body(w_hbm, d_hbm, idx_hbm, qvw_hbm, dq_hbm, z_hbm,
                     out0_hbm, out1_hbm, outd0_hbm, outd1_hbm,
                     acc, accd, idx_v, idxd_v, buf, dbuf):
    c = lax.axis_index("c")
    s = lax.axis_index("s")
    wid = c * SC_SUBCORES + s

    rows = pl.ds(pl.multiple_of(s * RPT, 8), RPT)
    rem = pl.ds(N - NREM, NREM)
    drows = pl.ds(pl.multiple_of(s * DPT, 8), DPT)
    last = s == SC_SUBCORES - 1

    @pl.when(c == 0)
    def _():
        pltpu.sync_copy(qvw_hbm.at[rows], acc.at[rows])
        pltpu.sync_copy(dq_hbm.at[drows], accd.at[drows])

    @pl.when(c == 1)
    def _():
        pltpu.sync_copy(z_hbm, acc.at[rows])
        pltpu.sync_copy(z_hbm.at[pl.ds(0, DPT)], accd.at[drows])

    @pl.when(jnp.logical_and(c == 0, last))
    def _():
        pltpu.sync_copy(qvw_hbm.at[rem], acc.at[rem])

    @pl.when(jnp.logical_and(c == 1, last))
    def _():
        pltpu.sync_copy(z_hbm.at[pl.ds(0, NREM)], acc.at[rem])

    plsc.subcore_barrier()

    base = pl.multiple_of(wid * EPT, 8)

    @pl.loop(0, NCHUNK)
    def _(i):
        off = pl.multiple_of(base + i * CHUNK, 8)
        pltpu.sync_copy(idx_hbm.at[pl.ds(off, CHUNK)], idx_v)
        pltpu.sync_copy(w_hbm.at[pl.ds(off, CHUNK)], buf)
        pltpu.sync_copy(d_hbm.at[pl.ds(off, CHUNK)], dbuf)
        for g in range(CHUNK // 16):
            idxd_v[pl.ds(g * 16, 16)] = lax.shift_right_logical(
                idx_v[pl.ds(g * 16, 16)], 4)
        pltpu.sync_copy(buf, acc.at[idx_v], add=True)
        pltpu.sync_copy(dbuf, accd.at[idxd_v], add=True)

    plsc.subcore_barrier()

    @pl.when(c == 0)
    def _():
        pltpu.sync_copy(acc.at[rows], out0_hbm.at[rows])
        pltpu.sync_copy(accd.at[drows], outd0_hbm.at[drows])

    @pl.when(c == 1)
    def _():
        pltpu.sync_copy(acc.at[rows], out1_hbm.at[rows])
        pltpu.sync_copy(accd.at[drows], outd1_hbm.at[drows])

    @pl.when(jnp.logical_and(c == 0, last))
    def _():
        pltpu.sync_copy(acc.at[rem], out0_hbm.at[rem])

    @pl.when(jnp.logical_and(c == 1, last))
    def _():
        pltpu.sync_copy(acc.at[rem], out1_hbm.at[rem])


def _sc_scatter(w128, den128, query_idx, qvw, dq640, zrows):
    k = pl.kernel(
        _sc_scatter_body,
        out_type=(
            jax.ShapeDtypeStruct((N, DIM), jnp.float32),
            jax.ShapeDtypeStruct((N, DIM), jnp.float32),
            jax.ShapeDtypeStruct((ND, DIM), jnp.float32),
            jax.ShapeDtypeStruct((ND, DIM), jnp.float32),
        ),
        mesh=_mesh(),
        compiler_params=_params(),
        scratch_types=[
            pltpu.VMEM_SHARED((N, DIM), jnp.float32),
            pltpu.VMEM_SHARED((ND, DIM), jnp.float32),
            pltpu.VMEM((CHUNK,), jnp.int32),
            pltpu.VMEM((CHUNK,), jnp.int32),
            pltpu.VMEM((CHUNK, DIM), jnp.float32),
            pltpu.VMEM((CHUNK, DIM), jnp.float32),
        ],
    )
    return k(w128, den128, query_idx, qvw, dq640, zrows)


def _finish_body(a0_ref, a1_ref, d0_ref, d1_ref, wp_ref, bp_ref, g_ref, b_ref,
                 o_ref):
    den = _mmn(d0_ref[...] + d1_ref[...], _selT())
    msg = (a0_ref[...] + a1_ref[...]) / den
    o = _mm(msg, wp_ref[...]) + bp_ref[...]
    m = jnp.mean(o, axis=-1, keepdims=True)
    v = jnp.mean((o - m) ** 2, axis=-1, keepdims=True)
    o_ref[...] = (o - m) * lax.rsqrt(v + 1e-5) * g_ref[...] + b_ref[...]


def _tc_finish(acc0, acc1, d0, d1, WP, bP, gamma, beta):
    full = lambda s: pl.BlockSpec(s, lambda i: (0, 0))
    return pl.pallas_call(
        _finish_body,
        grid=(N // BN,),
        in_specs=[
            pl.BlockSpec((BN, DIM), lambda i: (i, 0)),
            pl.BlockSpec((BN, DIM), lambda i: (i, 0)),
            pl.BlockSpec((BN, H), lambda i: (i, 0)),
            pl.BlockSpec((BN, H), lambda i: (i, 0)),
            full((DIM, DIM)), full((1, DIM)), full((1, DIM)), full((1, DIM)),
        ],
        out_specs=pl.BlockSpec((BN, DIM), lambda i: (i, 0)),
        out_shape=jax.ShapeDtypeStruct((N, DIM), jnp.float32),
    )(acc0, acc1, d0, d1, WP, bP, gamma, beta)


def kernel(query, keys, values, query_idx, WQ, bQ, WK, bK, WV, bV, WP, bP,
           a, prelu_w, gamma, beta):
    aflat = a.reshape(1, DIM)
    pw = prelu_w.reshape(1, 1)
    bQ2, bK2, bV2, bP2 = (x.reshape(1, DIM) for x in (bQ, bK, bV, bP))
    g2, b2 = gamma.reshape(1, DIM), beta.reshape(1, DIM)

    qQ, qvw, dq = _tc_node(query, WQ, WK, WV, bQ2, bK2, bV2, aflat, pw)
    qa = _sc_gather(qQ, query_idx)
    w128, den128 = _tc_edge(keys, values, qa, query_idx.reshape(E, 1),
                            WK, WV, bK2, bV2, aflat, pw)
    dq640 = jnp.concatenate(
        [dq.reshape(N * H // DIM, DIM),
         jnp.zeros((ND - N * H // DIM, DIM), jnp.float32)], axis=0)
    zrows = jnp.zeros((RPT, DIM), jnp.float32)
    acc0, acc1, dacc0, dacc1 = _sc_scatter(
        w128, den128, query_idx, qvw, dq640, zrows)
    d0 = dacc0.reshape(ND * SC_SUBCORES, H)[:N]
    d1 = dacc1.reshape(ND * SC_SUBCORES, H)[:N]
    return _tc_finish(acc0, acc1, d0, d1, WP, bP2, g2, b2)
